# baseline (speedup 1.0000x reference)
import functools

import jax
import jax.numpy as jnp
from jax import lax
from jax.experimental import pallas as pl
from jax.experimental.pallas import tpu as pltpu
from jax.experimental.pallas import tpu_sc as plsc

NUM_ROWS = 100000
FEAT_DIM = 138
N_SAMPLES = 16384
ID_DIM = 128
SHAPE_DIM = 10
TAIL_PAD = 16
LANES = 16

_NC = 2
_NS = 16
_NW = _NC * _NS
_BPW = N_SAMPLES // _NW
_CH = 128
_NCHUNK = _BPW // _CH
_FIRE = 16

_mesh = plsc.VectorSubcoreMesh(core_axis_name="c", subcore_axis_name="s")


@functools.partial(
    pl.kernel,
    mesh=_mesh,
    out_type=(
        jax.ShapeDtypeStruct((N_SAMPLES, ID_DIM), jnp.float32),
        jax.ShapeDtypeStruct((N_SAMPLES, SHAPE_DIM), jnp.float32),
    ),
    scratch_types=[
        pltpu.VMEM((_NCHUNK, _CH), jnp.int32),
        pltpu.VMEM((_BPW,), jnp.int32),
        pltpu.VMEM((2, _CH, ID_DIM), jnp.float32),
        pltpu.VMEM((_BPW * TAIL_PAD,), jnp.float32),
        pltpu.VMEM((_BPW, SHAPE_DIM), jnp.float32),
        pltpu.SemaphoreType.DMA,
        pltpu.SemaphoreType.DMA,
        pltpu.SemaphoreType.DMA,
    ],
    compiler_params=pltpu.CompilerParams(
        needs_layout_passes=False,
        disable_bounds_checks=True,
        disable_semaphore_checks=True,
    ),
)
def _gather_split(lead_hbm, tail_hbm, idx_hbm, id_hbm, shape_hbm,
                  idx_v, idx_vf, rows_v, tail_v, shape_v,
                  sem_g, sem_s, sem_w):
    wid = lax.axis_index("s") * _NC + lax.axis_index("c")
    base = wid * _BPW

    for j in range(_NCHUNK):
        pltpu.sync_copy(idx_hbm.at[pl.ds(base + j * _CH, _CH)], idx_v.at[j])
    pltpu.sync_copy(idx_hbm.at[pl.ds(base, _BPW)], idx_vf)

    def _tail_step(step, carry):
        off = step * _FIRE
        vec = idx_vf[pl.ds(off, _FIRE)]
        for t in range(_FIRE):
            r = vec[t]
            pltpu.async_copy(
                tail_hbm.at[pl.ds(r * TAIL_PAD, TAIL_PAD)],
                tail_v.at[pl.ds((off + t) * TAIL_PAD, TAIL_PAD)],
                sem_s)
        return carry

    lax.fori_loop(0, _BPW // _FIRE, _tail_step, 0)

    gathers = [None, None]
    wbs = [None, None]
    for j in range(_NCHUNK):
        s = j % 2
        if wbs[s] is not None:
            wbs[s].wait()
        gathers[s] = pltpu.async_copy(lead_hbm.at[idx_v.at[j]],
                                      rows_v.at[s], sem_g)
        if j > 0:
            p = (j - 1) % 2
            gathers[p].wait()
            wbs[p] = pltpu.async_copy(
                rows_v.at[p], id_hbm.at[pl.ds(base + (j - 1) * _CH, _CH)],
                sem_w)
    last = (_NCHUNK - 1) % 2
    gathers[last].wait()
    wbs[last] = pltpu.async_copy(
        rows_v.at[last], id_hbm.at[pl.ds(base + (_NCHUNK - 1) * _CH, _CH)],
        sem_w)

    pltpu.make_async_copy(tail_hbm.at[pl.ds(0, _BPW * TAIL_PAD)], tail_v,
                          sem_s).wait()

    lanes = lax.iota(jnp.int32, LANES)
    mask = lanes < SHAPE_DIM

    def _compact(i, carry):
        vals = tail_v[pl.ds(i * TAIL_PAD, LANES)]
        rows_idx = jnp.full((LANES,), i, dtype=jnp.int32)
        plsc.store_scatter(shape_v, [rows_idx, lanes], vals, mask=mask)
        return carry

    lax.fori_loop(0, _BPW, _compact, 0)

    pltpu.sync_copy(shape_v, shape_hbm.at[pl.ds(base, _BPW)])
    for wb in wbs:
        wb.wait()


def kernel(table, rand_id):
    lead = table[:, :ID_DIM]
    tail = jnp.pad(table[:, ID_DIM:], ((0, 0), (0, TAIL_PAD - SHAPE_DIM)))
    return _gather_split(lead, tail.reshape(-1), rand_id.astype(jnp.int32))


# speedup vs baseline: 1.9279x; 1.9279x over previous
import functools

import jax
import jax.numpy as jnp
from jax import lax
from jax.experimental import pallas as pl
from jax.experimental.pallas import tpu as pltpu
from jax.experimental.pallas import tpu_sc as plsc

NUM_ROWS = 100000
FEAT_DIM = 138
N_SAMPLES = 16384
ID_DIM = 128
SHAPE_DIM = 10

_NC = 2
_NS = 16
_NW = _NC * _NS
_BPW = N_SAMPLES // _NW
_CH = 128
_NCHUNK = _BPW // _CH
_FIRE = 16

_mesh = plsc.VectorSubcoreMesh(core_axis_name="c", subcore_axis_name="s")


@functools.partial(
    pl.kernel,
    mesh=_mesh,
    out_type=(
        jax.ShapeDtypeStruct((N_SAMPLES, ID_DIM), jnp.float32),
        jax.ShapeDtypeStruct((N_SAMPLES, SHAPE_DIM), jnp.float32),
    ),
    scratch_types=[
        pltpu.VMEM((_NCHUNK, _CH), jnp.int32),
        pltpu.VMEM((_BPW,), jnp.int32),
        pltpu.VMEM((3, _CH, ID_DIM), jnp.float32),
        pltpu.VMEM((_BPW, SHAPE_DIM), jnp.float32),
        pltpu.SemaphoreType.DMA,
        pltpu.SemaphoreType.DMA,
        pltpu.SemaphoreType.DMA,
        pltpu.SemaphoreType.DMA,
    ],
    compiler_params=pltpu.CompilerParams(
        disable_bounds_checks=True,
        disable_semaphore_checks=True,
        skip_device_barrier=True,
    ),
)
def _gather_split(table_hbm, idx_hbm, id_hbm, shape_hbm,
                  idx_v, idx_vf, rows_v, shape_v, sem_i, sem_g, sem_s, sem_w):
    wid = lax.axis_index("s") * _NC + lax.axis_index("c")
    base = wid * _BPW

    idx_copies = [
        pltpu.async_copy(idx_hbm.at[pl.ds(base + j * _CH, _CH)],
                         idx_v.at[j], sem_i)
        for j in range(_NCHUNK)
    ]
    idx_copies.append(
        pltpu.async_copy(idx_hbm.at[pl.ds(base, _BPW)], idx_vf, sem_i))
    for c in idx_copies:
        c.wait()

    def _shape_step(step, carry):
        off = step * _FIRE
        vec = idx_vf[pl.ds(off, _FIRE)]
        for t in range(_FIRE):
            r = vec[t]
            pltpu.async_copy(
                table_hbm.at[pl.ds(r, 1), pl.ds(ID_DIM, SHAPE_DIM)],
                shape_v.at[pl.ds(off + t, 1)],
                sem_s)
        return carry

    lax.fori_loop(0, _BPW // _FIRE, _shape_step, 0)

    id_src = table_hbm.at[:, pl.ds(0, ID_DIM)]

    def _fire(j, s):
        return pltpu.async_copy(id_src.at[idx_v.at[j]], rows_v.at[s], sem_g)

    def _wb(j, s):
        return pltpu.async_copy(
            rows_v.at[s], id_hbm.at[pl.ds(base + j * _CH, _CH)], sem_w)

    g0, g1, g2 = _fire(0, 0), _fire(1, 1), _fire(2, 2)
    g0.wait()
    wb0 = _wb(0, 0)
    g1.wait()
    wb1 = _wb(1, 1)
    wb0.wait()
    g3 = _fire(3, 0)
    g2.wait()
    wb2 = _wb(2, 2)
    g3.wait()
    wb3 = _wb(3, 0)
    wbs = [wb1, wb2, wb3]

    pltpu.make_async_copy(shape_hbm.at[pl.ds(base, _BPW)], shape_v,
                          sem_s).wait()
    pltpu.sync_copy(shape_v, shape_hbm.at[pl.ds(base, _BPW)])
    for wb in wbs:
        wb.wait()


def kernel(table, rand_id):
    return _gather_split(table, rand_id.astype(jnp.int32))
